# Initial kernel scaffold; baseline (speedup 1.0000x reference)
import jax
import jax.numpy as jnp
from jax.experimental import pallas as pl


def kernel(domain_idx, layer_idx, mu, sig, mu_table, sig_table, count, momentum_logit):
    raise NotImplementedError("write your pallas kernel here")


# speedup vs baseline: 1.2185x; 1.2185x over previous
import functools

import jax
import jax.numpy as jnp
from jax import lax
from jax.experimental import pallas as pl
from jax.experimental.pallas import tpu as pltpu
from jax.experimental.pallas import tpu_sc as plsc

ND = 16384
C = 1024
B = 16384
LAYERS = (0, 1)

NC = 2
NS = 16
L = 16

CS = 128
NCS = C // CS
DH = ND // 2
ACC_ROWS = DH + L
RPT = B // NS
SB = 256
NSB = RPT // SB
NG = SB // L
ZR = 64
DSTRIPE = DH // NS


def _sc_segment_sums(dom, mu2, sig2):
    mesh = plsc.VectorSubcoreMesh(core_axis_name="c", subcore_axis_name="s")
    out_type = (
        jax.ShapeDtypeStruct((ND, C), jnp.float32),
        jax.ShapeDtypeStruct((ND, C), jnp.float32),
        jax.ShapeDtypeStruct((ND, CS), jnp.float32),
    )
    scratch = [
        pltpu.VMEM((RPT,), jnp.int32),
        pltpu.VMEM((SB, CS), jnp.float32),
        pltpu.VMEM((L, CS), jnp.float32),
        pltpu.VMEM((ZR, CS), jnp.float32),
        pltpu.VMEM_SHARED((ACC_ROWS, CS), jnp.float32),
    ]

    @functools.partial(pl.kernel, out_type=out_type, mesh=mesh,
                       scratch_types=scratch)
    def k(dom_h, mu_h, sig_h, seg_mu_h, seg_sig_h, cnt_h,
          dom_v, stage, ones_v, zbuf, acc):
        c = lax.axis_index("c")
        s = lax.axis_index("s")

        zero16 = jnp.zeros((L,), jnp.float32)
        one16 = jnp.ones((L,), jnp.float32)

        def zb_outer(r, _):
            def zb_inner(j, _):
                zbuf[r, pl.ds(j * L, L)] = zero16
                return 0
            return lax.fori_loop(0, CS // L, zb_inner, 0)
        lax.fori_loop(0, ZR, zb_outer, 0)

        def ones_outer(r, _):
            def ones_inner(j, _):
                ones_v[r, pl.ds(j * L, L)] = one16
                return 0
            return lax.fori_loop(0, CS // L, ones_inner, 0)
        lax.fori_loop(0, L, ones_outer, 0)

        pltpu.sync_copy(dom_h.at[pl.ds(s * RPT, RPT)], dom_v)

        def zero_acc():
            def zb(i, _):
                pltpu.sync_copy(zbuf, acc.at[pl.ds(s * DSTRIPE + i * ZR, ZR)])
                return 0
            lax.fori_loop(0, DSTRIPE // ZR, zb, 0)

        def scatter_groups(dh_lo, src, base_row):
            lo_v = jnp.full((L,), dh_lo, jnp.int32)
            hi_v = jnp.full((L,), dh_lo + DH, jnp.int32)
            garb_v = jnp.full((L,), DH, jnp.int32)

            def g_body(g, _):
                d = dom_v[pl.ds(base_row + g * L, L)]
                m = (d >= lo_v) & (d < hi_v)
                svec = jnp.where(m, d - lo_v, garb_v)
                pltpu.sync_copy(src.at[pl.ds(g * L, L)], acc.at[svec],
                                add=True)
                return 0
            lax.fori_loop(0, NG, g_body, 0)

        def copy_out(dst, dh_lo, col):
            pltpu.sync_copy(
                acc.at[pl.ds(s * DSTRIPE, DSTRIPE)],
                dst.at[pl.ds(dh_lo + s * DSTRIPE, DSTRIPE),
                       pl.ds(col, CS)])

        for dh in range(2):
            @pl.when(c == dh)
            def _():
                dh_lo = dh * DH
                for src_h, dst_h in ((mu_h, seg_mu_h), (sig_h, seg_sig_h)):
                    for cs in range(NCS):
                        zero_acc()
                        plsc.subcore_barrier()
                        for sb in range(NSB):
                            pltpu.sync_copy(
                                src_h.at[pl.ds(s * RPT + sb * SB, SB),
                                         pl.ds(cs * CS, CS)],
                                stage)
                            scatter_groups(dh_lo, stage, sb * SB)
                        plsc.subcore_barrier()
                        copy_out(dst_h, dh_lo, cs * CS)
                        plsc.subcore_barrier()

        for dh in range(2):
            @pl.when(c == dh)
            def _():
                dh_lo = dh * DH
                zero_acc()
                plsc.subcore_barrier()
                lo_v = jnp.full((L,), dh_lo, jnp.int32)
                hi_v = jnp.full((L,), dh_lo + DH, jnp.int32)
                garb_v = jnp.full((L,), DH, jnp.int32)

                def cnt_group(g, _):
                    d = dom_v[pl.ds(g * L, L)]
                    m = (d >= lo_v) & (d < hi_v)
                    svec = jnp.where(m, d - lo_v, garb_v)
                    pltpu.sync_copy(ones_v, acc.at[svec], add=True)
                    return 0
                lax.fori_loop(0, RPT // L, cnt_group, 0)
                plsc.subcore_barrier()
                pltpu.sync_copy(
                    acc.at[pl.ds(s * DSTRIPE, DSTRIPE)],
                    cnt_h.at[pl.ds(dh_lo + s * DSTRIPE, DSTRIPE)])

    return k(dom, mu2, sig2)


BR = 256


def _tc_merge(seg_mu, seg_sig, cnt, mu_table, sig_table, count2d, mlog, upd):
    def body(mlog_ref, upd_ref, segmu_ref, segsig_ref, cnt_ref, mut_ref,
             sigt_ref, count_ref, muo_ref, sigo_ref, co_ref):
        m = jax.nn.sigmoid(mlog_ref[0, 0])
        u = upd_ref[0, 0] != 0
        bc = cnt_ref[:, 0:1]
        present = bc > 0.5
        upd_mask = jnp.logical_and(present, u)
        denom = jnp.maximum(bc, 1.0)
        cval = count_ref[...]
        first = cval == 0
        mu_mean = segmu_ref[...] / denom
        sig_mean = segsig_ref[...] / denom
        new_mu = jnp.where(first, mu_mean,
                           m * mut_ref[...] + (1.0 - m) * mu_mean)
        new_sig = jnp.where(first, sig_mean,
                            m * sigt_ref[...] + (1.0 - m) * sig_mean)
        muo_ref[...] = jnp.where(upd_mask, new_mu, mut_ref[...])
        sigo_ref[...] = jnp.where(upd_mask, new_sig, sigt_ref[...])
        co_ref[...] = cval + upd_mask.astype(jnp.int32)

    grid = (ND // BR,)
    return pl.pallas_call(
        body,
        grid=grid,
        in_specs=[
            pl.BlockSpec(memory_space=pltpu.SMEM),
            pl.BlockSpec(memory_space=pltpu.SMEM),
            pl.BlockSpec((BR, C), lambda i: (i, 0)),
            pl.BlockSpec((BR, C), lambda i: (i, 0)),
            pl.BlockSpec((BR, CS), lambda i: (i, 0)),
            pl.BlockSpec((BR, C), lambda i: (i, 0)),
            pl.BlockSpec((BR, C), lambda i: (i, 0)),
            pl.BlockSpec((BR, 1), lambda i: (i, 0)),
        ],
        out_specs=[
            pl.BlockSpec((BR, C), lambda i: (i, 0)),
            pl.BlockSpec((BR, C), lambda i: (i, 0)),
            pl.BlockSpec((BR, 1), lambda i: (i, 0)),
        ],
        out_shape=[
            jax.ShapeDtypeStruct((ND, C), jnp.float32),
            jax.ShapeDtypeStruct((ND, C), jnp.float32),
            jax.ShapeDtypeStruct((ND, 1), jnp.int32),
        ],
    )(mlog, upd, seg_mu, seg_sig, cnt, mu_table, sig_table, count2d)


def kernel(domain_idx, layer_idx, mu, sig, mu_table, sig_table, count,
           momentum_logit):
    mu2 = mu.reshape(B, C)
    sig2 = sig.reshape(B, C)
    dom = domain_idx.astype(jnp.int32)
    seg_mu, seg_sig, cnt = _sc_segment_sums(dom, mu2, sig2)
    upd = jnp.any(jnp.asarray(layer_idx) == jnp.asarray(LAYERS)) \
        .astype(jnp.int32).reshape(1, 1)
    mlog = jnp.asarray(momentum_logit, jnp.float32).reshape(1, 1)
    count2d = count.reshape(ND, 1)
    mu_f, sig_f, cnt_f = _tc_merge(seg_mu, seg_sig, cnt, mu_table, sig_table,
                                   count2d, mlog, upd)
    return mu_f, sig_f, cnt_f.reshape(ND)


# speedup vs baseline: 1.3810x; 1.1334x over previous
import functools

import jax
import jax.numpy as jnp
from jax import lax
from jax.experimental import pallas as pl
from jax.experimental.pallas import tpu as pltpu
from jax.experimental.pallas import tpu_sc as plsc

ND = 16384
C = 1024
B = 16384
LAYERS = (0, 1)

NC = 2
NS = 16
L = 16

CS = 128
NCS = C // CS
DH = ND // 2
ACC_ROWS = DH + L
RPT = B // NS
SB = 128
NSB = RPT // SB
GPB = SB // L
ZR = 64
DSTRIPE = DH // NS


def _sc_seg(dom, src, with_counts):
    mesh = plsc.VectorSubcoreMesh(core_axis_name="c", subcore_axis_name="s")
    out_type = [jax.ShapeDtypeStruct((ND, C), jnp.float32)]
    if with_counts:
        out_type.append(jax.ShapeDtypeStruct((ND, CS), jnp.float32))
    scratch = [
        pltpu.VMEM((RPT,), jnp.int32),
        pltpu.VMEM((SB, CS), jnp.float32),
        pltpu.VMEM((SB, CS), jnp.float32),
        pltpu.VMEM((L, CS), jnp.float32),
        pltpu.VMEM((ZR, CS), jnp.float32),
        pltpu.VMEM_SHARED((ACC_ROWS, CS), jnp.float32),
        pltpu.SemaphoreType.DMA,
        pltpu.SemaphoreType.DMA,
        pltpu.SemaphoreType.DMA,
        pltpu.SemaphoreType.DMA,
        pltpu.SemaphoreType.DMA,
        pltpu.SemaphoreType.DMA,
    ]

    @functools.partial(pl.kernel, out_type=tuple(out_type), mesh=mesh,
                       scratch_types=scratch)
    def k(dom_h, src_h, *refs):
        if with_counts:
            (seg_h, cnt_h, dom_v, stage0, stage1, ones_v, zbuf, acc,
             sem_st0, sem_st1, sem_sc0, sem_sc1, sem_cp, sem_z) = refs
        else:
            (seg_h, dom_v, stage0, stage1, ones_v, zbuf, acc,
             sem_st0, sem_st1, sem_sc0, sem_sc1, sem_cp, sem_z) = refs
            cnt_h = None
        c = lax.axis_index("c")
        s = lax.axis_index("s")
        stages = (stage0, stage1)
        sem_st = (sem_st0, sem_st1)
        sem_sc = (sem_sc0, sem_sc1)

        zero16 = jnp.zeros((L,), jnp.float32)
        one16 = jnp.ones((L,), jnp.float32)

        def zb_outer(r, _):
            def zb_inner(j, _):
                zbuf[r, pl.ds(j * L, L)] = zero16
                return 0
            return lax.fori_loop(0, CS // L, zb_inner, 0)
        lax.fori_loop(0, ZR, zb_outer, 0)

        if with_counts:
            def ones_outer(r, _):
                def ones_inner(j, _):
                    ones_v[r, pl.ds(j * L, L)] = one16
                    return 0
                return lax.fori_loop(0, CS // L, ones_inner, 0)
            lax.fori_loop(0, L, ones_outer, 0)

        pltpu.sync_copy(dom_h.at[pl.ds(s * RPT, RPT)], dom_v)

        dh_lo = c * DH
        lo_v = jnp.full((L,), dh_lo, jnp.int32)
        hi_v = jnp.full((L,), dh_lo + DH, jnp.int32)
        garb_v = jnp.full((L,), DH, jnp.int32)

        def zero_acc():
            def zf(i, _):
                pltpu.make_async_copy(
                    zbuf, acc.at[pl.ds(s * DSTRIPE + i * ZR, ZR)],
                    sem_z).start()
                return 0
            lax.fori_loop(0, DSTRIPE // ZR, zf, 0)

            def zw(i, _):
                pltpu.make_async_copy(
                    zbuf, acc.at[pl.ds(s * DSTRIPE, ZR)], sem_z).wait()
                return 0
            lax.fori_loop(0, DSTRIPE // ZR, zw, 0)

        def st_desc(sb, b, col):
            return pltpu.make_async_copy(
                src_h.at[pl.ds(s * RPT + sb * SB, SB), pl.ds(col, CS)],
                stages[b], sem_st[b])

        def fire_scatters(b, base_row):
            stg = stages[b]
            sem = sem_sc[b]

            def g_body(g, _):
                d = dom_v[pl.ds(base_row + g * L, L)]
                m = (d >= lo_v) & (d < hi_v)
                svec = jnp.where(m, d - lo_v, garb_v)
                pltpu.make_async_copy(
                    stg.at[pl.ds(g * L, L)], acc.at[svec],
                    sem).start(add=True)
                return 0
            lax.fori_loop(0, GPB, g_body, 0)

        def drain_sc(b):
            pltpu.make_async_copy(
                src_h.at[pl.ds(0, SB), pl.ds(0, CS)], stages[b],
                sem_sc[b]).wait()

        def do_pass(col):
            st_desc(0, 0, col).start()
            for sb in range(NSB):
                b = sb & 1
                st_desc(sb, b, col).wait()
                fire_scatters(b, sb * SB)
                if sb + 1 < NSB:
                    nb = 1 - b
                    if sb >= 1:
                        drain_sc(nb)
                    st_desc(sb + 1, nb, col).start()
            drain_sc(0)
            drain_sc(1)

        def copy_out_zero(col):
            cp = pltpu.make_async_copy(
                acc.at[pl.ds(s * DSTRIPE, DSTRIPE)],
                seg_h.at[pl.ds(dh_lo + s * DSTRIPE, DSTRIPE),
                         pl.ds(col, CS)],
                sem_cp)
            cp.start()
            cp.wait()
            zero_acc()

        zero_acc()
        plsc.subcore_barrier()

        def cs_body(csi, _):
            col = pl.multiple_of(csi * CS, CS)
            do_pass(col)
            plsc.subcore_barrier()
            copy_out_zero(col)
            plsc.subcore_barrier()
            return 0
        lax.fori_loop(0, NCS, cs_body, 0)

        if with_counts:
            def cnt_group(g, _):
                d = dom_v[pl.ds(g * L, L)]
                m = (d >= lo_v) & (d < hi_v)
                svec = jnp.where(m, d - lo_v, garb_v)
                pltpu.make_async_copy(
                    ones_v, acc.at[svec], sem_sc0).start(add=True)
                return 0
            lax.fori_loop(0, RPT // L, cnt_group, 0)

            def cnt_drain(i, _):
                drain_sc(0)
                return 0
            lax.fori_loop(0, (RPT // L) // GPB, cnt_drain, 0)
            plsc.subcore_barrier()
            pltpu.sync_copy(
                acc.at[pl.ds(s * DSTRIPE, DSTRIPE)],
                cnt_h.at[pl.ds(dh_lo + s * DSTRIPE, DSTRIPE)])

    return k(dom, src)


BR = 256


def _tc_merge(seg, cnt, table, count2d, mlog, upd, with_count_out):
    def body(mlog_ref, upd_ref, seg_ref, cnt_ref, tab_ref, count_ref,
             out_ref, *maybe_co):
        m = jax.nn.sigmoid(mlog_ref[0, 0])
        u = upd_ref[0, 0] != 0
        bc = cnt_ref[:, 0:1]
        present = bc > 0.5
        upd_mask = jnp.logical_and(present, u)
        denom = jnp.maximum(bc, 1.0)
        cval = count_ref[...]
        first = cval == 0
        mean = seg_ref[...] / denom
        new = jnp.where(first, mean, m * tab_ref[...] + (1.0 - m) * mean)
        out_ref[...] = jnp.where(upd_mask, new, tab_ref[...])
        if with_count_out:
            maybe_co[0][...] = cval + upd_mask.astype(jnp.int32)

    out_specs = [pl.BlockSpec((BR, C), lambda i: (i, 0))]
    out_shape = [jax.ShapeDtypeStruct((ND, C), jnp.float32)]
    if with_count_out:
        out_specs.append(pl.BlockSpec((BR, 1), lambda i: (i, 0)))
        out_shape.append(jax.ShapeDtypeStruct((ND, 1), jnp.int32))

    return pl.pallas_call(
        body,
        grid=(ND // BR,),
        in_specs=[
            pl.BlockSpec(memory_space=pltpu.SMEM),
            pl.BlockSpec(memory_space=pltpu.SMEM),
            pl.BlockSpec((BR, C), lambda i: (i, 0)),
            pl.BlockSpec((BR, CS), lambda i: (i, 0)),
            pl.BlockSpec((BR, C), lambda i: (i, 0)),
            pl.BlockSpec((BR, 1), lambda i: (i, 0)),
        ],
        out_specs=out_specs,
        out_shape=out_shape,
    )(mlog, upd, seg, cnt, table, count2d)


def kernel(domain_idx, layer_idx, mu, sig, mu_table, sig_table, count,
           momentum_logit):
    mu2 = mu.reshape(B, C)
    sig2 = sig.reshape(B, C)
    dom = domain_idx.astype(jnp.int32)
    seg_mu, cnt = _sc_seg(dom, mu2, with_counts=True)
    seg_sig, = _sc_seg(dom, sig2, with_counts=False)
    upd = jnp.any(jnp.asarray(layer_idx) == jnp.asarray(LAYERS)) \
        .astype(jnp.int32).reshape(1, 1)
    mlog = jnp.asarray(momentum_logit, jnp.float32).reshape(1, 1)
    count2d = count.reshape(ND, 1)
    mu_f, cnt_f = _tc_merge(seg_mu, cnt, mu_table, count2d, mlog, upd,
                            with_count_out=True)
    sig_f, = _tc_merge(seg_sig, cnt, sig_table, count2d, mlog, upd,
                       with_count_out=False)
    return mu_f, sig_f, cnt_f.reshape(ND)


# speedup vs baseline: 1.6764x; 1.2139x over previous
import functools

import jax
import jax.numpy as jnp
from jax import lax
from jax.experimental import pallas as pl
from jax.experimental.pallas import tpu as pltpu
from jax.experimental.pallas import tpu_sc as plsc

ND = 16384
C = 1024
B = 16384
LAYERS = (0, 1)

NC = 2
NS = 16
L = 16

CS = 128
NCS = C // CS
DH = ND // 2
ACC_ROWS = DH + L
RPT = B // NS
SB = 128
NSB = RPT // SB
GPB = SB // L
ZR = 64
DSTRIPE = DH // NS


def _sc_seg(dom, src, with_counts):
    mesh = plsc.VectorSubcoreMesh(core_axis_name="c", subcore_axis_name="s")
    out_type = [jax.ShapeDtypeStruct((ND, C), jnp.float32)]
    if with_counts:
        out_type.append(jax.ShapeDtypeStruct((ND, CS), jnp.float32))
    scratch = [
        pltpu.VMEM((RPT,), jnp.int32),
        pltpu.VMEM((SB, CS), jnp.float32),
        pltpu.VMEM((SB, CS), jnp.float32),
        pltpu.VMEM((L, CS), jnp.float32),
        pltpu.VMEM((ZR, CS), jnp.float32),
        pltpu.VMEM_SHARED((ACC_ROWS, CS), jnp.float32),
        pltpu.SemaphoreType.DMA,
        pltpu.SemaphoreType.DMA,
        pltpu.SemaphoreType.DMA,
        pltpu.SemaphoreType.DMA,
        pltpu.SemaphoreType.DMA,
        pltpu.SemaphoreType.DMA,
    ]

    @functools.partial(pl.kernel, out_type=tuple(out_type), mesh=mesh,
                       scratch_types=scratch)
    def k(dom_h, src_h, *refs):
        if with_counts:
            (seg_h, cnt_h, dom_v, stage0, stage1, ones_v, zbuf, acc,
             sem_st0, sem_st1, sem_sc0, sem_sc1, sem_cp, sem_z) = refs
        else:
            (seg_h, dom_v, stage0, stage1, ones_v, zbuf, acc,
             sem_st0, sem_st1, sem_sc0, sem_sc1, sem_cp, sem_z) = refs
            cnt_h = None
        c = lax.axis_index("c")
        s = lax.axis_index("s")
        stages = (stage0, stage1)
        sem_st = (sem_st0, sem_st1)
        sem_sc = (sem_sc0, sem_sc1)

        zero16 = jnp.zeros((L,), jnp.float32)
        one16 = jnp.ones((L,), jnp.float32)

        def zb_outer(r, _):
            def zb_inner(j, _):
                zbuf[r, pl.ds(j * L, L)] = zero16
                return 0
            return lax.fori_loop(0, CS // L, zb_inner, 0)
        lax.fori_loop(0, ZR, zb_outer, 0)

        if with_counts:
            def ones_outer(r, _):
                def ones_inner(j, _):
                    ones_v[r, pl.ds(j * L, L)] = one16
                    return 0
                return lax.fori_loop(0, CS // L, ones_inner, 0)
            lax.fori_loop(0, L, ones_outer, 0)

        pltpu.sync_copy(dom_h.at[pl.ds(s * RPT, RPT)], dom_v)

        dh_lo = c * DH
        lo_v = jnp.full((L,), dh_lo, jnp.int32)
        hi_v = jnp.full((L,), dh_lo + DH, jnp.int32)
        garb_v = jnp.full((L,), DH, jnp.int32)

        def zero_acc():
            def zf(i, _):
                pltpu.make_async_copy(
                    zbuf, acc.at[pl.ds(s * DSTRIPE + i * ZR, ZR)],
                    sem_z).start()
                return 0
            lax.fori_loop(0, DSTRIPE // ZR, zf, 0)

            def zw(i, _):
                pltpu.make_async_copy(
                    zbuf, acc.at[pl.ds(s * DSTRIPE, ZR)], sem_z).wait()
                return 0
            lax.fori_loop(0, DSTRIPE // ZR, zw, 0)

        lane8 = jax.lax.iota(jnp.int32, L) * NCS

        def fire_gathers(b, base_row, csi):
            stg = stages[b]

            def g_body(g, _):
                rb = (s * RPT + base_row + g * L) * NCS + csi
                idx = lane8 + jnp.full((L,), rb, jnp.int32)
                pltpu.make_async_copy(
                    src_h.at[idx], stg.at[pl.ds(g * L, L)],
                    sem_st[b]).start()
                return 0
            lax.fori_loop(0, GPB, g_body, 0)

        def wait_gathers(b):
            pltpu.make_async_copy(
                src_h.at[pl.ds(0, SB)], stages[b], sem_st[b]).wait()

        def fire_scatters(b, base_row):
            stg = stages[b]
            sem = sem_sc[b]

            def g_body(g, _):
                d = dom_v[pl.ds(base_row + g * L, L)]
                m = (d >= lo_v) & (d < hi_v)
                svec = jnp.where(m, d - lo_v, garb_v)
                pltpu.make_async_copy(
                    stg.at[pl.ds(g * L, L)], acc.at[svec],
                    sem).start(add=True)
                return 0
            lax.fori_loop(0, GPB, g_body, 0)

        def drain_sc(b):
            pltpu.make_async_copy(
                src_h.at[pl.ds(0, SB)], stages[b], sem_sc[b]).wait()

        def do_pass(csi):
            fire_gathers(0, 0, csi)
            for sb in range(NSB):
                b = sb & 1
                wait_gathers(b)
                fire_scatters(b, sb * SB)
                if sb + 1 < NSB:
                    nb = 1 - b
                    if sb >= 1:
                        drain_sc(nb)
                    fire_gathers(nb, (sb + 1) * SB, csi)
            drain_sc(0)
            drain_sc(1)

        def copy_out_zero(col):
            cp = pltpu.make_async_copy(
                acc.at[pl.ds(s * DSTRIPE, DSTRIPE)],
                seg_h.at[pl.ds(dh_lo + s * DSTRIPE, DSTRIPE),
                         pl.ds(col, CS)],
                sem_cp)
            cp.start()
            cp.wait()
            zero_acc()

        zero_acc()
        plsc.subcore_barrier()

        def cs_body(csi, _):
            do_pass(csi)
            plsc.subcore_barrier()
            copy_out_zero(pl.multiple_of(csi * CS, CS))
            plsc.subcore_barrier()
            return 0
        lax.fori_loop(0, NCS, cs_body, 0)

        if with_counts:
            def cnt_group(g, _):
                d = dom_v[pl.ds(g * L, L)]
                m = (d >= lo_v) & (d < hi_v)
                svec = jnp.where(m, d - lo_v, garb_v)
                pltpu.make_async_copy(
                    ones_v, acc.at[svec], sem_sc0).start(add=True)
                return 0
            lax.fori_loop(0, RPT // L, cnt_group, 0)

            def cnt_drain(i, _):
                drain_sc(0)
                return 0
            lax.fori_loop(0, (RPT // L) // GPB, cnt_drain, 0)
            plsc.subcore_barrier()
            pltpu.sync_copy(
                acc.at[pl.ds(s * DSTRIPE, DSTRIPE)],
                cnt_h.at[pl.ds(dh_lo + s * DSTRIPE, DSTRIPE)])

    return k(dom, src)


BR = 256


def _tc_merge(seg, cnt, table, count2d, mlog, upd, with_count_out):
    def body(mlog_ref, upd_ref, seg_ref, cnt_ref, tab_ref, count_ref,
             out_ref, *maybe_co):
        m = jax.nn.sigmoid(mlog_ref[0, 0])
        u = upd_ref[0, 0] != 0
        bc = cnt_ref[:, 0:1]
        present = bc > 0.5
        upd_mask = jnp.logical_and(present, u)
        denom = jnp.maximum(bc, 1.0)
        cval = count_ref[...]
        first = cval == 0
        mean = seg_ref[...] / denom
        new = jnp.where(first, mean, m * tab_ref[...] + (1.0 - m) * mean)
        out_ref[...] = jnp.where(upd_mask, new, tab_ref[...])
        if with_count_out:
            maybe_co[0][...] = cval + upd_mask.astype(jnp.int32)

    out_specs = [pl.BlockSpec((BR, C), lambda i: (i, 0))]
    out_shape = [jax.ShapeDtypeStruct((ND, C), jnp.float32)]
    if with_count_out:
        out_specs.append(pl.BlockSpec((BR, 1), lambda i: (i, 0)))
        out_shape.append(jax.ShapeDtypeStruct((ND, 1), jnp.int32))

    return pl.pallas_call(
        body,
        grid=(ND // BR,),
        in_specs=[
            pl.BlockSpec(memory_space=pltpu.SMEM),
            pl.BlockSpec(memory_space=pltpu.SMEM),
            pl.BlockSpec((BR, C), lambda i: (i, 0)),
            pl.BlockSpec((BR, CS), lambda i: (i, 0)),
            pl.BlockSpec((BR, C), lambda i: (i, 0)),
            pl.BlockSpec((BR, 1), lambda i: (i, 0)),
        ],
        out_specs=out_specs,
        out_shape=out_shape,
    )(mlog, upd, seg, cnt, table, count2d)


def kernel(domain_idx, layer_idx, mu, sig, mu_table, sig_table, count,
           momentum_logit):
    mu2 = mu.reshape(B * NCS, CS)
    sig2 = sig.reshape(B * NCS, CS)
    dom = domain_idx.astype(jnp.int32)
    seg_mu, cnt = _sc_seg(dom, mu2, with_counts=True)
    seg_sig, = _sc_seg(dom, sig2, with_counts=False)
    upd = jnp.any(jnp.asarray(layer_idx) == jnp.asarray(LAYERS)) \
        .astype(jnp.int32).reshape(1, 1)
    mlog = jnp.asarray(momentum_logit, jnp.float32).reshape(1, 1)
    count2d = count.reshape(ND, 1)
    mu_f, cnt_f = _tc_merge(seg_mu, cnt, mu_table, count2d, mlog, upd,
                            with_count_out=True)
    sig_f, = _tc_merge(seg_sig, cnt, sig_table, count2d, mlog, upd,
                       with_count_out=False)
    return mu_f, sig_f, cnt_f.reshape(ND)
